# baseline (speedup 1.0000x reference)
import functools

import jax
import jax.numpy as jnp
from jax import lax
from jax.experimental import pallas as pl
from jax.experimental.pallas import tpu as pltpu
from jax.experimental.pallas import tpu_sc as plsc

_DIM = 64
_NUM_WORKERS = 32
_MEGA = 8


def _build(batch: int, seq: int):
    rows_per_worker = batch // _NUM_WORKERS
    n_mega = rows_per_worker // _MEGA
    n_pairs = n_mega // 2
    mesh = plsc.VectorSubcoreMesh(core_axis_name="c", subcore_axis_name="s")

    @functools.partial(
        pl.kernel,
        mesh=mesh,
        compiler_params=pltpu.CompilerParams(use_tc_tiling_on_sc=False),
        out_type=jax.ShapeDtypeStruct((batch, seq, _DIM), jnp.float32),
        scratch_types=[
            pltpu.VMEM((rows_per_worker, seq), jnp.int32),
            pltpu.VMEM((_MEGA, seq, _DIM), jnp.float32),
            pltpu.VMEM((_MEGA, seq, _DIM), jnp.float32),
            pltpu.SemaphoreType.DMA,
            pltpu.SemaphoreType.DMA,
        ],
    )
    def gather_kernel(idx_hbm, table_hbm, out_hbm, idx_v, buf_a, buf_b, sem_a, sem_b):
        wid = lax.axis_index("s") * 2 + lax.axis_index("c")
        base = wid * rows_per_worker
        pltpu.sync_copy(idx_hbm.at[pl.ds(base, rows_per_worker)], idx_v)

        def start_mega(m, buf, sem):
            for c in range(_MEGA):
                pltpu.async_copy(
                    table_hbm.at[idx_v.at[m * _MEGA + c]],
                    buf.at[c],
                    sem,
                )

        def wait_mega(buf, sem):
            pltpu.make_async_copy(
                out_hbm.at[pl.ds(base, _MEGA)], buf, sem
            ).wait()

        def out_mega(m, buf):
            pltpu.sync_copy(
                buf, out_hbm.at[pl.ds(base + m * _MEGA, _MEGA)]
            )

        start_mega(0, buf_a, sem_a)

        def body(t, carry):
            start_mega(2 * t + 1, buf_b, sem_b)
            wait_mega(buf_a, sem_a)
            out_mega(2 * t, buf_a)
            start_mega(2 * t + 2, buf_a, sem_a)
            wait_mega(buf_b, sem_b)
            out_mega(2 * t + 1, buf_b)
            return carry

        lax.fori_loop(0, n_pairs - 1, body, 0)
        start_mega(n_mega - 1, buf_b, sem_b)
        wait_mega(buf_a, sem_a)
        out_mega(n_mega - 2, buf_a)
        wait_mega(buf_b, sem_b)
        out_mega(n_mega - 1, buf_b)

    return gather_kernel


def kernel(x, table):
    batch, seq = x.shape
    return _build(batch, seq)(x, table)
